# baseline (speedup 1.0000x reference)
import functools

import jax
import jax.numpy as jnp
from jax import lax
from jax.experimental import pallas as pl
from jax.experimental.pallas import tpu as pltpu
from jax.experimental.pallas import tpu_sc as plsc

_B = 16384
_F = 26
_FIELD = 40000
_NW = 32
_BPW = _B // _NW
_L = 16
_CH = 128
_NJV = _BPW // _L
_NCC = _BPW // _CH


def _sc_body(x_hbm, table_hbm, bias_hbm, out_hbm, xv, idxv, vals, outv,
             biasv, sem):
  nc = lax.axis_index("c")
  ns = lax.axis_index("s")
  wid = ns * 2 + nc
  base = wid * _BPW

  pltpu.sync_copy(x_hbm.at[:, pl.ds(base, _BPW)], xv)
  pltpu.sync_copy(bias_hbm, biasv)

  def add_offsets(p, _):
    f = p // _NJV
    jv = p - f * _NJV
    off = f * _FIELD
    idxv[f, pl.ds(jv * _L, _L)] = xv[f, pl.ds(jv * _L, _L)] + off
    return 0

  lax.fori_loop(0, _F * _NJV, add_offsets, 0)

  def gather(c, _):
    f = c // _NCC
    cc = c - f * _NCC
    pltpu.async_copy(
        table_hbm.at[idxv.at[f, pl.ds(cc * _CH, _CH)]],
        vals.at[f, pl.ds(cc * _CH, _CH)],
        sem,
    ).wait()
    return 0

  lax.fori_loop(0, _F * _NCC, gather, 0)

  bvec = biasv[pl.ds(0, _L)]

  def reduce(jv, _):
    sl = pl.ds(jv * _L, _L)
    acc = vals[0, sl] + bvec
    for f in range(1, _F):
      acc = acc + vals[f, sl]
    outv[sl] = acc
    return 0

  lax.fori_loop(0, _NJV, reduce, 0)

  pltpu.sync_copy(outv, out_hbm.at[pl.ds(base, _BPW)])


@jax.jit
def kernel(x, table, bias):
  xt = x.astype(jnp.int32).T
  table1 = table.reshape(-1)
  bias16 = jnp.broadcast_to(bias.astype(jnp.float32), (_L,))

  mesh = plsc.VectorSubcoreMesh(
      core_axis_name="c", subcore_axis_name="s", num_cores=2, num_subcores=16)
  out = pl.kernel(
      _sc_body,
      out_type=jax.ShapeDtypeStruct((_B,), jnp.float32),
      mesh=mesh,
      scratch_types=[
          pltpu.VMEM((_F, _BPW), jnp.int32),
          pltpu.VMEM((_F, _BPW), jnp.int32),
          pltpu.VMEM((_F, _BPW), jnp.float32),
          pltpu.VMEM((_BPW,), jnp.float32),
          pltpu.VMEM((_L,), jnp.float32),
          pltpu.SemaphoreType.DMA,
      ],
  )(xt, table1, bias16)
  return out.reshape(_B, 1)


# speedup vs baseline: 1.6289x; 1.6289x over previous
import jax
import jax.numpy as jnp
from jax import lax
from jax.experimental import pallas as pl
from jax.experimental.pallas import tpu as pltpu
from jax.experimental.pallas import tpu_sc as plsc

_B = 16384
_F = 26
_FIELD = 40000
_NW = 32
_BPW = _B // _NW
_L = 16
_CH = 128
_NJV = _BPW // _L
_NCC = _BPW // _CH


def _sc_body(x_hbm, table_hbm, bias_hbm, out_hbm, xv, idxv, vals, outv,
             biasv, sem):
  nc = lax.axis_index("c")
  ns = lax.axis_index("s")
  wid = ns * 2 + nc
  base = wid * _BPW

  pltpu.sync_copy(x_hbm.at[:, pl.ds(base, _BPW)], xv)
  pltpu.sync_copy(bias_hbm, biasv)

  def compute_and_fire(f):
    off = f * _FIELD
    fbase = f * _BPW
    for jv in range(_NJV):
      idxv[pl.ds(fbase + jv * _L, _L)] = xv[f, pl.ds(jv * _L, _L)] + off
    for cc in range(_NCC):
      pltpu.async_copy(
          table_hbm.at[idxv.at[pl.ds(fbase + cc * _CH, _CH)]],
          vals.at[pl.ds(fbase + cc * _CH, _CH)],
          sem,
      )

  def drain_one_field():
    pltpu.make_async_copy(
        table_hbm.at[pl.ds(0, _BPW)], vals.at[pl.ds(0, _BPW)], sem
    ).wait()

  compute_and_fire(0)

  def pipe(f, _):
    compute_and_fire(f)
    drain_one_field()
    return 0

  lax.fori_loop(1, _F, pipe, 0)
  drain_one_field()

  bvec = biasv[pl.ds(0, _L)]

  def reduce(jv, _):
    j16 = jv * _L
    acc = vals[pl.ds(j16, _L)] + bvec
    for f in range(1, _F):
      acc = acc + vals[pl.ds(f * _BPW + j16, _L)]
    outv[pl.ds(j16, _L)] = acc
    return 0

  lax.fori_loop(0, _NJV, reduce, 0)

  pltpu.sync_copy(outv, out_hbm.at[pl.ds(base, _BPW)])


@jax.jit
def kernel(x, table, bias):
  xt = x.astype(jnp.int32).T
  table1 = table.reshape(-1)
  bias16 = jnp.broadcast_to(bias.astype(jnp.float32), (_L,))

  mesh = plsc.VectorSubcoreMesh(
      core_axis_name="c", subcore_axis_name="s", num_cores=2, num_subcores=16)
  out = pl.kernel(
      _sc_body,
      out_type=jax.ShapeDtypeStruct((_B,), jnp.float32),
      mesh=mesh,
      scratch_types=[
          pltpu.VMEM((_F, _BPW), jnp.int32),
          pltpu.VMEM((_F * _BPW,), jnp.int32),
          pltpu.VMEM((_F * _BPW,), jnp.float32),
          pltpu.VMEM((_BPW,), jnp.float32),
          pltpu.VMEM((_L,), jnp.float32),
          pltpu.SemaphoreType.DMA,
      ],
  )(xt, table1, bias16)
  return out.reshape(_B, 1)


# speedup vs baseline: 1.8224x; 1.1188x over previous
import jax
import jax.numpy as jnp
from jax import lax
from jax.experimental import pallas as pl
from jax.experimental.pallas import tpu as pltpu
from jax.experimental.pallas import tpu_sc as plsc

_B = 16384
_F = 26
_FH = _F // 2
_FIELD = 40000
_NW = 32
_BPW = _B // _NW
_L = 16
_NJV = _BPW // _L
_HALF = _FH * _BPW


def _sc_body(x_hbm, table_hbm, bias_hbm, out_hbm, xv, idxv, vals, outv,
             biasv, sem):
  nc = lax.axis_index("c")
  ns = lax.axis_index("s")
  wid = ns * 2 + nc
  base = wid * _BPW

  pltpu.sync_copy(x_hbm.at[:, pl.ds(base, _BPW)], xv)
  pltpu.sync_copy(bias_hbm, biasv)

  def compute_idx(flo, fhi):
    def body(f, _):
      off = f * _FIELD
      fbase = f * _BPW
      for jv in range(_NJV):
        idxv[pl.ds(fbase + jv * _L, _L)] = xv[f, pl.ds(jv * _L, _L)] + off
      return 0

    lax.fori_loop(flo, fhi, body, 0)

  def fire(flo):
    sl = pl.ds(flo * _BPW, _HALF)
    return pltpu.async_copy(table_hbm.at[idxv.at[sl]], vals.at[sl], sem)

  def accumulate(flo, fhi, first):
    def body(jv, _):
      j16 = jv * _L
      sl = pl.ds(j16, _L)
      head = biasv[pl.ds(0, _L)] if first else outv[sl]
      acc = head + vals[pl.ds(flo * _BPW + j16, _L)]
      for f in range(flo + 1, fhi):
        acc = acc + vals[pl.ds(f * _BPW + j16, _L)]
      outv[sl] = acc
      return 0

    lax.fori_loop(0, _NJV, body, 0)

  compute_idx(0, _FH)
  copy_a = fire(0)
  compute_idx(_FH, _F)
  copy_b = fire(_FH)
  copy_a.wait()
  accumulate(0, _FH, True)
  copy_b.wait()
  accumulate(_FH, _F, False)

  pltpu.sync_copy(outv, out_hbm.at[pl.ds(base, _BPW)])


@jax.jit
def kernel(x, table, bias):
  xt = x.astype(jnp.int32).T
  table1 = table.reshape(-1)
  bias16 = jnp.broadcast_to(bias.astype(jnp.float32), (_L,))

  mesh = plsc.VectorSubcoreMesh(
      core_axis_name="c", subcore_axis_name="s", num_cores=2, num_subcores=16)
  out = pl.kernel(
      _sc_body,
      out_type=jax.ShapeDtypeStruct((_B,), jnp.float32),
      mesh=mesh,
      scratch_types=[
          pltpu.VMEM((_F, _BPW), jnp.int32),
          pltpu.VMEM((_F * _BPW,), jnp.int32),
          pltpu.VMEM((_F * _BPW,), jnp.float32),
          pltpu.VMEM((_BPW,), jnp.float32),
          pltpu.VMEM((_L,), jnp.float32),
          pltpu.SemaphoreType.DMA,
      ],
  )(xt, table1, bias16)
  return out.reshape(_B, 1)


# speedup vs baseline: 2.0181x; 1.1074x over previous
import jax
import jax.numpy as jnp
from jax import lax
from jax.experimental import pallas as pl
from jax.experimental.pallas import tpu as pltpu
from jax.experimental.pallas import tpu_sc as plsc

_B = 16384
_F = 26
_FIELD = 40000
_NT = 16
_NPAIR = _F // 2
_BPC = _B // 2
_L = 16
_BPT = _BPC // _NT
_NJV = _BPC // _L


def _sc_body(x_hbm, table_hbm, bias_hbm, out_hbm,
             ta, tb, xa, xb, partial, red, outv, biasv, shared, sem):
  nc = lax.axis_index("c")
  ns = lax.axis_index("s")
  cbase = nc * _BPC

  @pl.when(ns < _NPAIR)
  def gather_phase():
    fa = 2 * ns
    fb = fa + 1
    cps = [
        pltpu.async_copy(table_hbm.at[pl.ds(fa * _FIELD, _FIELD)], ta, sem),
        pltpu.async_copy(table_hbm.at[pl.ds(fb * _FIELD, _FIELD)], tb, sem),
        pltpu.async_copy(x_hbm.at[pl.ds(fa * _B + cbase, _BPC)], xa, sem),
        pltpu.async_copy(x_hbm.at[pl.ds(fb * _B + cbase, _BPC)], xb, sem),
    ]
    for cp in cps:
      cp.wait()

    def body(j, _):
      sl = pl.ds(j * _L, _L)
      va = plsc.load_gather(ta, [xa[sl]])
      vb = plsc.load_gather(tb, [xb[sl]])
      partial[sl] = va + vb
      return 0

    lax.fori_loop(0, _NJV, body, 0)
    pltpu.sync_copy(partial, shared.at[pl.ds(ns * _BPC, _BPC)])

  plsc.subcore_barrier()

  pltpu.sync_copy(bias_hbm, biasv)
  cps = [
      pltpu.async_copy(
          shared.at[pl.ds(t * _BPC + ns * _BPT, _BPT)],
          red.at[pl.ds(t * _BPT, _BPT)],
          sem,
      )
      for t in range(_NPAIR)
  ]
  for cp in cps:
    cp.wait()

  def reduce(jv, _):
    sl = pl.ds(jv * _L, _L)
    j16 = jv * _L
    acc = biasv[pl.ds(0, _L)] + red[pl.ds(j16, _L)]
    for t in range(1, _NPAIR):
      acc = acc + red[pl.ds(t * _BPT + j16, _L)]
    outv[sl] = acc
    return 0

  lax.fori_loop(0, _BPT // _L, reduce, 0)
  pltpu.sync_copy(outv, out_hbm.at[pl.ds(cbase + ns * _BPT, _BPT)])


@jax.jit
def kernel(x, table, bias):
  xt = x.astype(jnp.int32).T.reshape(-1)
  table1 = table.reshape(-1)
  bias16 = jnp.broadcast_to(bias.astype(jnp.float32), (_L,))

  mesh = plsc.VectorSubcoreMesh(
      core_axis_name="c", subcore_axis_name="s", num_cores=2, num_subcores=16)
  out = pl.kernel(
      _sc_body,
      out_type=jax.ShapeDtypeStruct((_B,), jnp.float32),
      mesh=mesh,
      compiler_params=pltpu.CompilerParams(needs_layout_passes=False),
      scratch_types=[
          pltpu.VMEM((_FIELD,), jnp.float32),
          pltpu.VMEM((_FIELD,), jnp.float32),
          pltpu.VMEM((_BPC,), jnp.int32),
          pltpu.VMEM((_BPC,), jnp.int32),
          pltpu.VMEM((_BPC,), jnp.float32),
          pltpu.VMEM((_NPAIR * _BPT,), jnp.float32),
          pltpu.VMEM((_BPT,), jnp.float32),
          pltpu.VMEM((_L,), jnp.float32),
          pltpu.VMEM_SHARED((_NPAIR * _BPC,), jnp.float32),
          pltpu.SemaphoreType.DMA,
      ],
  )(xt, table1, bias16)
  return out.reshape(_B, 1)


# speedup vs baseline: 3.8425x; 1.9040x over previous
import jax
import jax.numpy as jnp
from jax import lax
from jax.experimental import pallas as pl
from jax.experimental.pallas import tpu as pltpu
from jax.experimental.pallas import tpu_sc as plsc

_B = 16384
_F = 26
_FIELD = 40000
_NT = 16
_NPAIR = _F // 2
_BPC = _B // 2
_L = 16
_BPT = _BPC // _NT
_NJV = _BPC // _L
_TW = 128
_RPP = 2 * _FIELD // _TW


def _sc_body(x_hbm, table_hbm, bias_hbm, out_hbm,
             tblk, xa, xb, partial, red, outv, biasv, shared, sem):
  nc = lax.axis_index("c")
  ns = lax.axis_index("s")
  cbase = nc * _BPC

  @pl.when(ns < _NPAIR)
  def gather_phase():
    fa = 2 * ns
    fb = fa + 1
    row_lo = ns * _RPP
    cps = [
        pltpu.async_copy(table_hbm.at[pl.ds(row_lo, _RPP), :], tblk, sem),
        pltpu.async_copy(x_hbm.at[pl.ds(fa * _B + cbase, _BPC)], xa, sem),
        pltpu.async_copy(x_hbm.at[pl.ds(fb * _B + cbase, _BPC)], xb, sem),
    ]
    for cp in cps:
      cp.wait()

    def body(j, _):
      sl = pl.ds(j * _L, _L)
      ia = xa[sl]
      ib = xb[sl] + _FIELD
      va = plsc.load_gather(tblk, [ia >> 7, ia & 127])
      vb = plsc.load_gather(tblk, [ib >> 7, ib & 127])
      partial[sl] = va + vb
      return 0

    lax.fori_loop(0, _NJV, body, 0)
    pltpu.sync_copy(partial, shared.at[pl.ds(ns * _BPC, _BPC)])

  plsc.subcore_barrier()

  pltpu.sync_copy(bias_hbm, biasv)
  cps = [
      pltpu.async_copy(
          shared.at[pl.ds(t * _BPC + ns * _BPT, _BPT)],
          red.at[pl.ds(t * _BPT, _BPT)],
          sem,
      )
      for t in range(_NPAIR)
  ]
  for cp in cps:
    cp.wait()

  def reduce(jv, _):
    sl = pl.ds(jv * _L, _L)
    j16 = jv * _L
    acc = biasv[pl.ds(0, _L)] + red[pl.ds(j16, _L)]
    for t in range(1, _NPAIR):
      acc = acc + red[pl.ds(t * _BPT + j16, _L)]
    outv[sl] = acc
    return 0

  lax.fori_loop(0, _BPT // _L, reduce, 0)
  pltpu.sync_copy(outv, out_hbm.at[pl.ds(cbase + ns * _BPT, _BPT)])


@jax.jit
def kernel(x, table, bias):
  xt = x.astype(jnp.int32).T.reshape(-1)
  tpad = jnp.pad(table.astype(jnp.float32), ((0, 1040384 - _F * _FIELD), (0, 0)))
  t2 = tpad.reshape(-1).reshape(1040384 // _TW, _TW)
  bias16 = jnp.broadcast_to(bias.astype(jnp.float32), (_L,))

  mesh = plsc.VectorSubcoreMesh(
      core_axis_name="c", subcore_axis_name="s", num_cores=2, num_subcores=16)
  out = pl.kernel(
      _sc_body,
      out_type=jax.ShapeDtypeStruct((_B,), jnp.float32),
      mesh=mesh,
      compiler_params=pltpu.CompilerParams(
          needs_layout_passes=False, use_tc_tiling_on_sc=False),
      scratch_types=[
          pltpu.VMEM((_RPP, _TW), jnp.float32),
          pltpu.VMEM((_BPC,), jnp.int32),
          pltpu.VMEM((_BPC,), jnp.int32),
          pltpu.VMEM((_BPC,), jnp.float32),
          pltpu.VMEM((_NPAIR * _BPT,), jnp.float32),
          pltpu.VMEM((_BPT,), jnp.float32),
          pltpu.VMEM((_L,), jnp.float32),
          pltpu.VMEM_SHARED((_NPAIR * _BPC,), jnp.float32),
          pltpu.SemaphoreType.DMA,
      ],
  )(xt, t2, bias16)
  return out.reshape(_B, 1)


# speedup vs baseline: 4.0010x; 1.0413x over previous
import jax
import jax.numpy as jnp
from jax import lax
from jax.experimental import pallas as pl
from jax.experimental.pallas import tpu as pltpu
from jax.experimental.pallas import tpu_sc as plsc

_B = 16384
_F = 26
_FIELD = 40000
_NT = 16
_NPAIR = _F // 2
_BPC = _B // 2
_L = 16
_BPT = _BPC // _NT
_NJV = _BPC // _L
_TW = 128
_RPP = 2 * _FIELD // _TW
_NPAD = 1040384


def _sc_body(x_hbm, table_hbm, bias_hbm, out_hbm,
             tblk, xab, partial, red, outv, biasv, shared, sem):
  nc = lax.axis_index("c")
  ns = lax.axis_index("s")
  cbase = nc * _BPC
  zero16 = jnp.zeros((_L,), jnp.int32)

  @pl.when(ns < _NPAIR)
  def gather_phase():
    fa = 2 * ns
    row_lo = ns * _RPP
    cps = [
        pltpu.async_copy(table_hbm.at[pl.ds(row_lo, _RPP), :], tblk, sem),
        pltpu.async_copy(
            x_hbm.at[pl.ds(fa, 2), pl.ds(cbase, _BPC)], xab, sem),
    ]
    for cp in cps:
      cp.wait()

    def body(j, _):
      sl = pl.ds(j * _L, _L)
      ia = xab[0, sl]
      ib = xab[1, sl] + _FIELD
      va = plsc.load_gather(tblk, [ia >> 7, ia & 127])
      vb = plsc.load_gather(tblk, [ib >> 7, ib & 127])
      partial[sl] = va + vb
      return 0

    lax.fori_loop(0, _NJV, body, 0)
    pltpu.sync_copy(partial, shared.at[pl.ds(ns * _BPC, _BPC)])

  plsc.subcore_barrier()

  pltpu.sync_copy(bias_hbm, biasv.at[pl.ds(0, 1)])
  cps = [
      pltpu.async_copy(
          shared.at[pl.ds(t * _BPC + ns * _BPT, _BPT)],
          red.at[pl.ds(t * _BPT, _BPT)],
          sem,
      )
      for t in range(_NPAIR)
  ]
  for cp in cps:
    cp.wait()
  bvec = plsc.load_gather(biasv, [zero16])

  def reduce(jv, _):
    sl = pl.ds(jv * _L, _L)
    j16 = jv * _L
    acc = bvec + red[pl.ds(j16, _L)]
    for t in range(1, _NPAIR):
      acc = acc + red[pl.ds(t * _BPT + j16, _L)]
    outv[sl] = acc
    return 0

  lax.fori_loop(0, _BPT // _L, reduce, 0)
  pltpu.sync_copy(outv, out_hbm.at[pl.ds(cbase + ns * _BPT, _BPT)])


@jax.jit
def kernel(x, table, bias):
  xt = x.astype(jnp.int32).T
  tpad = jnp.pad(table.astype(jnp.float32), ((0, _NPAD - _F * _FIELD), (0, 0)))
  t2 = tpad.reshape(-1).reshape(_NPAD // _TW, _TW)

  mesh = plsc.VectorSubcoreMesh(
      core_axis_name="c", subcore_axis_name="s", num_cores=2, num_subcores=16)
  out = pl.kernel(
      _sc_body,
      out_type=jax.ShapeDtypeStruct((_B,), jnp.float32),
      mesh=mesh,
      compiler_params=pltpu.CompilerParams(
          needs_layout_passes=False, use_tc_tiling_on_sc=False),
      scratch_types=[
          pltpu.VMEM((_RPP, _TW), jnp.float32),
          pltpu.VMEM((2, _BPC), jnp.int32),
          pltpu.VMEM((_BPC,), jnp.float32),
          pltpu.VMEM((_NPAIR * _BPT,), jnp.float32),
          pltpu.VMEM((_BPT,), jnp.float32),
          pltpu.VMEM((_L,), jnp.float32),
          pltpu.VMEM_SHARED((_NPAIR * _BPC,), jnp.float32),
          pltpu.SemaphoreType.DMA,
      ],
  )(xt, t2, bias.astype(jnp.float32))
  return out.reshape(_B, 1)
